# baseline (speedup 1.0000x reference)
import jax
import jax.numpy as jnp
from jax import lax
from jax.experimental import pallas as pl
from jax.experimental.pallas import tpu as pltpu
from jax.experimental.pallas import tpu_sc as plsc

_E = 320000
_N0 = 10000
_F = 16
_NC = 2
_NS = 16
_NW = _NC * _NS
_EW = _E // _NW
_CH = 80
_NCHUNK = _EW // _CH

_LVL = []
_n = _N0
while _n > 2:
    _LVL.append(_n)
    _n = (_n + 1) // 2
_NLVL = len(_LVL)
_N3 = _LVL[3]
_SEG3 = (_N3 + _F - 1) // _F
_HROWS = _N3 * _SEG3


def _pad_rows(n):
    return ((n + 1 + _NS - 1) // _NS) * _NS


_mesh = plsc.VectorSubcoreMesh(core_axis_name="c", subcore_axis_name="s")
_SC_PARAMS = pltpu.CompilerParams(use_tc_tiling_on_sc=False,
                                  needs_layout_passes=False)


def _fori(n, body):
    lax.fori_loop(0, n, lambda i, c: (body(i), 0)[1], 0)


_NBUF = 5


def _sweeps(level, with_cnt):
    n = _LVL[level]
    rows_budget = 2900 if with_cnt else 5800
    s = 1
    while n // s > rows_budget:
        s *= 2
    return s


def _mk_sc_agg(level, with_cnt):
    n = _LVL[level]
    npad = ((n + 7) // 8) * 8
    S = _sweeps(level, with_cnt)
    nloc = n // S
    rows = nloc + 8
    grp = _CH // _F
    nbuf = 5 if with_cnt else 10

    def body(src_h, dst_h, x_h, *refs):
        if with_cnt:
            (out_agg, out_cnt, srclv, dstv, ring, agg1, cnt1,
             *sems) = refs
        else:
            (out_agg, srclv, dstv, ring, agg1, *sems) = refs
            cnt1 = out_cnt = None
        c = lax.axis_index("c")
        s = lax.axis_index("s")
        wid = s * _NC + c
        base = wid * _EW
        pltpu.sync_copy(src_h.at[pl.ds(base, _EW)], srclv)
        pltpu.sync_copy(dst_h.at[pl.ds(base, _EW)], dstv)
        if level:
            def shl(i):
                sl_ = pl.ds(i * _F, _F)
                srclv[sl_] = lax.shift_right_logical(srclv[sl_], level)
            _fori(_EW // _F, shl)

        zero16 = jnp.zeros((_F,), jnp.float32)
        one16 = jnp.full((_F,), 1.0, jnp.float32)
        iot = lax.iota(jnp.int32, _F)
        trash16 = jnp.full((_F,), nloc * _F, jnp.int32)

        def fire(j, b):
            pltpu.async_copy(x_h.at[srclv.at[pl.ds(j * _CH, _CH)]],
                             ring.at[b], sems[b])

        def wait(j, b):
            pltpu.make_async_copy(
                x_h.at[srclv.at[pl.ds(j * _CH, _CH)]],
                ring.at[b], sems[b]).wait()

        def sweep(sw, _):
            lo = sw * nloc

            def zr(i):
                agg1[pl.ds(i * _F, _F)] = zero16
                if with_cnt:
                    cnt1[pl.ds(i * _F, _F)] = zero16
            _fori(rows, zr)

            def proc(j, b):
                wait(j, b)
                for g in range(grp):
                    e0 = g * _F
                    dv = dstv[pl.ds(j * _CH + e0, _F)]
                    if level:
                        dl = lax.shift_right_logical(dv, level)
                        slv = srclv[pl.ds(j * _CH + e0, _F)]
                        bad = slv == dl
                    else:
                        dl = dv
                        bad = None
                    dloc = dl - lo
                    oob = (dloc < 0) | (dloc >= nloc)
                    bad = oob if bad is None else (bad | oob)
                    rb = jnp.where(bad, trash16, dloc * _F)
                    for k in range(_F):
                        idx = rb[k] + iot
                        plsc.addupdate_scatter(
                            agg1, [idx], ring[b, e0 + k])
                        if with_cnt:
                            plsc.addupdate_scatter(cnt1, [idx], one16)

            for b in range(nbuf):
                fire(b, b)
            full = _NCHUNK // nbuf

            def chunk_iter(t, _2):
                for b in range(nbuf):
                    j = t * nbuf + b
                    proc(j, b)
                    @pl.when(j + nbuf < _NCHUNK)
                    def _f():
                        fire(j + nbuf, b)
                return _2
            lax.fori_loop(0, full, chunk_iter, 0)
            for b in range(_NCHUNK - full * nbuf):
                proc(full * nbuf + b, b)

            pltpu.sync_copy(agg1.at[pl.ds(0, nloc * _F)],
                            out_agg.at[wid, pl.ds(lo * _F, nloc * _F)])
            if with_cnt:
                pltpu.sync_copy(cnt1.at[pl.ds(0, nloc * _F)],
                                out_cnt.at[wid, pl.ds(lo * _F, nloc * _F)])
            return _
        lax.fori_loop(0, S, sweep, 0)
        if npad > n:
            pz = (npad - n) * _F
            pltpu.sync_copy(agg1.at[pl.ds((nloc + 1) * _F, pz)],
                            out_agg.at[wid, pl.ds(n * _F, pz)])
            if with_cnt:
                pltpu.sync_copy(cnt1.at[pl.ds((nloc + 1) * _F, pz)],
                                out_cnt.at[wid, pl.ds(n * _F, pz)])

    shp = jax.ShapeDtypeStruct((_NW, npad * _F), jnp.float32)
    out_type = [shp, shp] if with_cnt else shp
    scratch = [
        pltpu.VMEM((_EW,), jnp.int32),
        pltpu.VMEM((_EW,), jnp.int32),
        pltpu.VMEM((nbuf, _CH, _F), jnp.float32),
        pltpu.VMEM((rows * _F,), jnp.float32),
    ]
    if with_cnt:
        scratch.append(pltpu.VMEM((rows * _F,), jnp.float32))
    scratch += [pltpu.SemaphoreType.DMA] * nbuf
    return pl.kernel(body, mesh=_mesh, out_type=out_type,
                     scratch_types=scratch, compiler_params=_SC_PARAMS)


_HHALF = _HROWS // 2
_EW2 = _E // _NS
_NCHUNK2 = _EW2 // _CH


def _mk_sc_hist():
    n_trash = _pad_rows(_HHALF)
    rpt = n_trash // _NS
    nzd = rpt // _CH
    rem = rpt % _CH
    grp = _CH // _F

    def body(src_h, dst_h, out_h, srcv, dstv, row2, lane2, rowsv, ash,
             gsem):
        c = lax.axis_index("c")
        s = lax.axis_index("s")
        base = s * _EW2
        pltpu.sync_copy(src_h.at[pl.ds(base, _EW2)], srcv)
        pltpu.sync_copy(dst_h.at[pl.ds(base, _EW2)], dstv)

        zero16 = jnp.zeros((_F,), jnp.float32)
        _fori(_CH, lambda i: rowsv.__setitem__(i, zero16))
        tb = s * rpt
        _fori(nzd, lambda k: pltpu.sync_copy(
            rowsv, ash.at[pl.ds(tb + k * _CH, _CH)]))
        if rem:
            pltpu.sync_copy(rowsv.at[pl.ds(0, rem)],
                            ash.at[pl.ds(tb + nzd * _CH, rem)])
        plsc.subcore_barrier()

        trash = jnp.full((_F,), _HHALF, jnp.int32)
        dlo = c * (_N3 // 2)

        def ib(i):
            sv = lax.shift_right_logical(srcv[pl.ds(i * _F, _F)], 3)
            dv = lax.shift_right_logical(dstv[pl.ds(i * _F, _F)], 3)
            dr = dv - dlo
            row = dr * _SEG3 + lax.shift_right_logical(sv, 4)
            bad = (sv == dv) | (dr < 0) | (dr >= _N3 // 2)
            row = jnp.where(bad, trash, row)
            r = i // grp
            co = (i % grp) * _F
            row2[r, pl.ds(co, _F)] = row
            lane2[r, pl.ds(co, _F)] = jnp.bitwise_and(sv, _F - 1)
        _fori(_EW2 // _F, ib)

        iot = lax.iota(jnp.int32, _F)

        def cb(j):
            for g in range(grp):
                lv = lane2[j, pl.ds(g * _F, _F)]
                for k in range(_F):
                    rowsv[g * _F + k] = jnp.where(iot == lv[k], 1.0, 0.0)
            pltpu.sync_copy(rowsv, ash.at[row2.at[j]], add=True)
        _fori(_NCHUNK2, cb)
        plsc.subcore_barrier()

        _fori(nzd, lambda k: pltpu.sync_copy(
            ash.at[pl.ds(tb + k * _CH, _CH)],
            out_h.at[c, pl.ds(tb + k * _CH, _CH)]))
        if rem:
            tl = pl.ds(tb + nzd * _CH, rem)
            pltpu.sync_copy(ash.at[tl], out_h.at[c, tl])

    return pl.kernel(
        body, mesh=_mesh,
        out_type=jax.ShapeDtypeStruct((_NC, n_trash, _F), jnp.float32),
        scratch_types=[
            pltpu.VMEM((_EW2,), jnp.int32),
            pltpu.VMEM((_EW2,), jnp.int32),
            pltpu.VMEM((_NCHUNK2, _CH), jnp.int32),
            pltpu.VMEM((_NCHUNK2, _CH), jnp.int32),
            pltpu.VMEM((_CH, _F), jnp.float32),
            pltpu.VMEM_SHARED((n_trash, _F), jnp.float32),
            pltpu.SemaphoreType.DMA,
        ], compiler_params=_SC_PARAMS)


def _dot(a, b):
    return jax.lax.dot_general(a, b, (((1,), (0,)), ((), ())),
                               precision=lax.Precision.HIGHEST,
                               preferred_element_type=jnp.float32)


def _tc_reduce(parts, n):
    npad = ((n + 7) // 8) * 8
    n2 = npad * _F // 128
    rb = min(n2, 256)
    nb = -(-n2 // rb)
    ins = [p.reshape(_NW, n2, 128) for p in parts]
    spec = pl.BlockSpec((_NW, rb, 128), lambda i: (0, i, 0))
    ospec = pl.BlockSpec((rb, 128), lambda i: (i, 0))

    def body(*refs):
        k = len(ins)
        for i in range(k):
            refs[k + i][...] = jnp.sum(refs[i][...], axis=0)

    outs = pl.pallas_call(
        body, grid=(nb,), in_specs=[spec] * len(ins),
        out_specs=[ospec] * len(ins),
        out_shape=[jax.ShapeDtypeStruct((n2, 128), jnp.float32)] * len(ins),
    )(*ins)
    return [o.reshape(npad, _F)[:n] for o in outs]


def _blk(n):
    br = n if n <= 2500 else 2000
    return br, -(-n // br)


def _tc_update(agg, cnt, x, w, *, stack2=None, mlp=None):
    n = x.shape[0]
    br, nb = _blk(n)
    ix = lambda i: (i, 0)
    row16 = pl.BlockSpec((br, _F), ix)
    row32 = pl.BlockSpec((br, 2 * _F), ix)
    wsp = pl.BlockSpec((_F, _F), lambda i: (0, 0))
    bsp = pl.BlockSpec((1, _F), lambda i: (0, 0))
    wlt, bl, wrt = w

    args = [agg, cnt, x, wlt, bl, wrt]
    specs = [row16, row16, row16, wsp, bsp, wsp]
    out_shape = [jax.ShapeDtypeStruct((n, _F), jnp.float32)]
    out_specs = [row16]
    if stack2 is not None:
        args.append(stack2)
        specs.append(row32)
        out_shape.append(jax.ShapeDtypeStruct((n, 2 * _F), jnp.float32))
        out_specs.append(row32)
    if mlp is not None:
        for wm, bm in mlp[:3]:
            args += [wm, bm]
            specs += [wsp, bsp]
        wf, bf = mlp[3]
        args += [wf, bf]
        specs += [pl.BlockSpec((_F, 1), lambda i: (0, 0)),
                  pl.BlockSpec((1, 1), lambda i: (0, 0))]
        out_shape.append(jax.ShapeDtypeStruct((n, 1), jnp.float32))
        out_specs.append(pl.BlockSpec((br, 1), ix))

    def kbody(*refs):
        aggr, cntr = refs[0][...], refs[1][...]
        k = 2
        xr, wltr, blr, wrtr = refs[k:k + 4]
        k += 4
        mean = aggr / jnp.maximum(cntr[:, 0:1], 1.0)
        y = jnp.tanh(_dot(mean, wltr[...]) + blr[...] +
                     _dot(xr[...], wrtr[...]))
        outs = [y]
        if stack2 is not None:
            st = refs[k][...]
            k += 1
            outs.append((jnp.concatenate([y, y], axis=1) + st) * 0.5)
        if mlp is not None:
            h = y
            for _ in range(3):
                h = jnp.tanh(_dot(h, refs[k][...]) + refs[k + 1][...])
                k += 2
            outs.append(_dot(h, refs[k][...]) + refs[k + 1][...])
            k += 2
        for o, r in zip(outs, refs[k:]):
            r[...] = o

    res = pl.pallas_call(
        kbody, grid=(nb,), in_specs=specs, out_specs=out_specs,
        out_shape=out_shape,
    )(*args)
    return res


def _tc_pool(y, n):
    y2 = y.reshape(n // 2, 2 * _F)
    br, nb = _blk(n // 2)

    def body(yr, outr):
        v = yr[...]
        outr[...] = (v[:, :_F] + v[:, _F:]) * 0.5

    return pl.pallas_call(
        body, grid=(nb,),
        in_specs=[pl.BlockSpec((br, 2 * _F), lambda i: (i, 0))],
        out_specs=pl.BlockSpec((br, _F), lambda i: (i, 0)),
        out_shape=jax.ShapeDtypeStruct((n // 2, _F), jnp.float32),
    )(y2)


def _iota_pool_mats(nc, n):
    rc = lax.broadcasted_iota(jnp.int32, (nc, n), 0)
    cc = lax.broadcasted_iota(jnp.int32, (nc, n), 1)
    P = (cc // 2 == rc).astype(jnp.float32)
    rr = lax.broadcasted_iota(jnp.int32, (n, nc), 0)
    cr = lax.broadcasted_iota(jnp.int32, (n, nc), 1)
    PT = (rr // 2 == cr).astype(jnp.float32)
    return P, PT


def _tc_deep(x3, hmat, stack2, pre_w, post_w, coarse_w):
    def body(x3r, hr, st2r, *wr):
        ws = [w[...] for w in wr[:15]]
        outr = wr[15]
        pre = [tuple(ws[0:3]), tuple(ws[3:6])]
        post = [tuple(ws[6:9]), tuple(ws[9:12])]
        co = tuple(ws[12:15])

        A = hr[...][:, :_N3]
        x = x3r[...]

        def conv(x, A, w):
            wlt, bl, wrt = w
            cnt = jnp.maximum(jnp.sum(A, axis=1, keepdims=True), 1.0)
            mean = _dot(A, x) / cnt
            return jnp.tanh(_dot(mean, wlt) + bl + _dot(x, wrt))

        stacks = []
        amats = []
        for l in range(3, _NLVL):
            n = _LVL[l]
            for i in range(2):
                x = conv(x, A, pre[i])
            stacks.append(x)
            amats.append(A)
            nc = (n + 1) // 2
            P, PT = _iota_pool_mats(nc, n)
            sizes = jnp.where(
                lax.broadcasted_iota(jnp.int32, (nc, 1), 0) * 2 + 1 < n,
                2.0, 1.0)
            x = _dot(P, x) / sizes
            An = _dot(_dot(P, A), PT)
            ri = lax.broadcasted_iota(jnp.int32, (nc, nc), 0)
            ci = lax.broadcasted_iota(jnp.int32, (nc, nc), 1)
            A = jnp.where(ri == ci, 0.0, An)
        x = conv(x, A, co)
        for l in range(_NLVL - 1, 2, -1):
            n = _LVL[l]
            nc = (n + 1) // 2
            _, PT = _iota_pool_mats(nc, n)
            x = (_dot(PT, x) + stacks.pop()) * 0.5
            A = amats.pop()
            for i in range(2):
                x = conv(x, A, post[i])
        outr[...] = (jnp.concatenate([x, x], axis=1) + st2r[...]) * 0.5

    hw = _SEG3 * _F

    def full(s):
        return pl.BlockSpec(s, lambda: tuple(0 for _ in s))

    wargs = []
    wspecs = []
    for trip in (*pre_w, *post_w, coarse_w):
        for a in trip:
            wargs.append(a)
            wspecs.append(full(a.shape))
    return pl.pallas_call(
        body,
        in_specs=[full((_N3, _F)), full((_N3, hw)),
                  full((_N3, 2 * _F))] + wspecs,
        out_specs=full((_N3, 2 * _F)),
        out_shape=jax.ShapeDtypeStruct((_N3, 2 * _F), jnp.float32),
    )(x3, hmat, stack2, *wargs)


def _prep_w(tr):
    wl, bl, wr = tr
    return (wl.T, bl.reshape(1, _F), wr.T)


def kernel(x, params, edge_index, batch):
    src = edge_index[0]
    dst = edge_index[1]
    x16 = jnp.pad(x, ((0, 0), (0, _F - 1)))

    wl_f, bl_f, wr_f = params['first']
    first_w = (jnp.pad(wl_f.T, ((0, _F - 1), (0, 0))),
               bl_f.reshape(1, _F),
               jnp.pad(wr_f.T, ((0, _F - 1), (0, 0))))
    pre_w = [_prep_w(t) for t in params['pre']]
    post_w = [_prep_w(t) for t in params['post']]
    coarse_w = _prep_w(params['coarse'])
    mlp_w = [(params[nm][0].T, params[nm][1].reshape(1, _F))
             for nm in ('lin1', 'lin2', 'lin3')]
    wf, bf = params['final']
    mlp_w.append((wf.T, bf.reshape(1, 1)))

    aggk = {l: _mk_sc_agg(l, False) for l in (0, 1, 2)}
    aggck = {l: _mk_sc_agg(l, True) for l in (0, 1, 2)}

    hist = _mk_sc_hist()(src, dst)
    hmat = jnp.concatenate(
        [hist[0, :_HHALF], hist[1, :_HHALF]], axis=0
    ).reshape(_N3, _SEG3 * _F)

    aggp, cntp = aggck[0](src, dst, x16)
    agg16, cnt0 = _tc_reduce([aggp, cntp], _N0)
    (cur,) = _tc_update(agg16, cnt0, x16, first_w)

    stacks = []
    cnts = {0: cnt0}
    for l in range(3):
        n = _LVL[l]
        for i in range(2):
            if l > 0 and i == 0:
                aggp, cntp = aggck[l](src, dst, cur)
                agg16, cnts[l] = _tc_reduce([aggp, cntp], n)
            else:
                aggp = aggk[l](src, dst, cur)
                (agg16,) = _tc_reduce([aggp], n)
            (cur,) = _tc_update(agg16, cnts[l], cur, pre_w[i])
        stacks.append(cur)
        cur = _tc_pool(cur, n)

    up = _tc_deep(cur, hmat, stacks[2].reshape(_N3, 2 * _F),
                  pre_w, post_w, coarse_w)
    cur = up.reshape(_LVL[2], _F)

    for l in (2, 1, 0):
        aggp = aggk[l](src, dst, cur)
        (agg16,) = _tc_reduce([aggp], _LVL[l])
        (cur,) = _tc_update(agg16, cnts[l], cur, post_w[0])
        aggp = aggk[l](src, dst, cur)
        (agg16,) = _tc_reduce([aggp], _LVL[l])
        if l > 0:
            st2 = stacks[l - 1].reshape(_LVL[l], 2 * _F)
            _, up = _tc_update(agg16, cnts[l], cur, post_w[1], stack2=st2)
            cur = up.reshape(_LVL[l - 1], _F)
        else:
            _, out = _tc_update(agg16, cnts[l], cur, post_w[1], mlp=mlp_w)
    return out


# speedup vs baseline: 1.1643x; 1.1643x over previous
import jax
import jax.numpy as jnp
from jax import lax
from jax.experimental import pallas as pl
from jax.experimental.pallas import tpu as pltpu
from jax.experimental.pallas import tpu_sc as plsc

_E = 320000
_N0 = 10000
_F = 16
_NC = 2
_NS = 16
_NW = _NC * _NS
_EW = _E // _NW
_CH = 80
_NCHUNK = _EW // _CH

_LVL = []
_n = _N0
while _n > 2:
    _LVL.append(_n)
    _n = (_n + 1) // 2
_NLVL = len(_LVL)
_N3 = _LVL[3]
_SEG3 = (_N3 + _F - 1) // _F
_HROWS = _N3 * _SEG3


def _pad_rows(n):
    return ((n + 1 + _NS - 1) // _NS) * _NS


_mesh = plsc.VectorSubcoreMesh(core_axis_name="c", subcore_axis_name="s")
_SC_PARAMS = pltpu.CompilerParams(use_tc_tiling_on_sc=False)


def _fori(n, body):
    lax.fori_loop(0, n, lambda i, c: (body(i), 0)[1], 0)


def _mk_sc_agg(level, n, with_cnt):
    n_trash = _pad_rows(n)
    rpt = n_trash // _NS
    nzd = rpt // _CH
    rem = rpt % _CH

    def body(src_h, dst_h, x_h, *refs):
        if with_cnt:
            (out_agg, out_cnt, srcv, dstv, srcl2, dstl2, rowsv, constv,
             aggsh, cntsh, gsem) = refs
        else:
            (out_agg, srcv, dstv, srcl2, dstl2, rowsv, constv,
             aggsh, gsem) = refs
            cntsh = out_cnt = None
        c = lax.axis_index("c")
        s = lax.axis_index("s")
        wid = s * _NC + c
        base = wid * _EW
        pltpu.sync_copy(src_h.at[pl.ds(base, _EW)], srcv)
        pltpu.sync_copy(dst_h.at[pl.ds(base, _EW)], dstv)

        zero16 = jnp.zeros((_F,), jnp.float32)
        _fori(_CH, lambda i: constv.__setitem__(i, zero16))
        tb = s * rpt

        def zdma(k):
            pltpu.sync_copy(constv, aggsh.at[pl.ds(tb + k * _CH, _CH)])
            if with_cnt:
                pltpu.sync_copy(constv, cntsh.at[pl.ds(tb + k * _CH, _CH)])
        _fori(nzd, zdma)
        if rem:
            tl = pl.ds(tb + nzd * _CH, rem)
            pltpu.sync_copy(constv.at[pl.ds(0, rem)], aggsh.at[tl])
            if with_cnt:
                pltpu.sync_copy(constv.at[pl.ds(0, rem)], cntsh.at[tl])
        if with_cnt:
            one16 = jnp.full((_F,), 1.0, jnp.float32)
            _fori(_CH, lambda i: constv.__setitem__(i, one16))
        plsc.subcore_barrier()

        trash = jnp.full((_F,), n, jnp.int32)

        def ib(i):
            sv = srcv[pl.ds(i * _F, _F)]
            dv = dstv[pl.ds(i * _F, _F)]
            if level:
                sl = lax.shift_right_logical(sv, level)
                dl = lax.shift_right_logical(dv, level)
                dl = jnp.where(sl == dl, trash, dl)
            else:
                sl, dl = sv, dv
            r = i // (_CH // _F)
            co = (i % (_CH // _F)) * _F
            srcl2[r, pl.ds(co, _F)] = sl
            dstl2[r, pl.ds(co, _F)] = dl
        _fori(_EW // _F, ib)

        def cb(j):
            pltpu.async_copy(x_h.at[srcl2.at[j]], rowsv, gsem).wait()
            pltpu.sync_copy(rowsv, aggsh.at[dstl2.at[j]], add=True)
            if with_cnt:
                pltpu.sync_copy(constv, cntsh.at[dstl2.at[j]], add=True)
        _fori(_NCHUNK, cb)
        plsc.subcore_barrier()

        def co_(k):
            sl_ = pl.ds(tb + k * _CH, _CH)
            pltpu.sync_copy(aggsh.at[sl_], out_agg.at[c, sl_])
            if with_cnt:
                pltpu.sync_copy(cntsh.at[sl_], out_cnt.at[c, sl_])
        _fori(nzd, co_)
        if rem:
            tl = pl.ds(tb + nzd * _CH, rem)
            pltpu.sync_copy(aggsh.at[tl], out_agg.at[c, tl])
            if with_cnt:
                pltpu.sync_copy(cntsh.at[tl], out_cnt.at[c, tl])

    shp = jax.ShapeDtypeStruct((_NC, n_trash, _F), jnp.float32)
    out_type = [shp, shp] if with_cnt else shp
    scratch = [
        pltpu.VMEM((_EW,), jnp.int32),
        pltpu.VMEM((_EW,), jnp.int32),
        pltpu.VMEM((_NCHUNK, _CH), jnp.int32),
        pltpu.VMEM((_NCHUNK, _CH), jnp.int32),
        pltpu.VMEM((_CH, _F), jnp.float32),
        pltpu.VMEM((_CH, _F), jnp.float32),
        pltpu.VMEM_SHARED((n_trash, _F), jnp.float32),
        pltpu.SemaphoreType.DMA,
    ]
    if with_cnt:
        scratch.insert(7, pltpu.VMEM_SHARED((n_trash, _F), jnp.float32))
    return pl.kernel(body, mesh=_mesh, out_type=out_type,
                     scratch_types=scratch, compiler_params=_SC_PARAMS)


_HHALF = _HROWS // 2
_EW2 = _E // _NS
_NCHUNK2 = _EW2 // _CH


def _mk_sc_hist():
    n_trash = _pad_rows(_HHALF)
    rpt = n_trash // _NS
    nzd = rpt // _CH
    rem = rpt % _CH
    grp = _CH // _F

    def body(src_h, dst_h, out_h, srcv, dstv, row2, lane2, rowsv, ash,
             gsem):
        c = lax.axis_index("c")
        s = lax.axis_index("s")
        base = s * _EW2
        pltpu.sync_copy(src_h.at[pl.ds(base, _EW2)], srcv)
        pltpu.sync_copy(dst_h.at[pl.ds(base, _EW2)], dstv)

        zero16 = jnp.zeros((_F,), jnp.float32)
        _fori(_CH, lambda i: rowsv.__setitem__(i, zero16))
        tb = s * rpt
        _fori(nzd, lambda k: pltpu.sync_copy(
            rowsv, ash.at[pl.ds(tb + k * _CH, _CH)]))
        if rem:
            pltpu.sync_copy(rowsv.at[pl.ds(0, rem)],
                            ash.at[pl.ds(tb + nzd * _CH, rem)])
        plsc.subcore_barrier()

        trash = jnp.full((_F,), _HHALF, jnp.int32)
        dlo = c * (_N3 // 2)

        def ib(i):
            sv = lax.shift_right_logical(srcv[pl.ds(i * _F, _F)], 3)
            dv = lax.shift_right_logical(dstv[pl.ds(i * _F, _F)], 3)
            dr = dv - dlo
            row = dr * _SEG3 + lax.shift_right_logical(sv, 4)
            bad = (sv == dv) | (dr < 0) | (dr >= _N3 // 2)
            row = jnp.where(bad, trash, row)
            r = i // grp
            co = (i % grp) * _F
            row2[r, pl.ds(co, _F)] = row
            lane2[r, pl.ds(co, _F)] = jnp.bitwise_and(sv, _F - 1)
        _fori(_EW2 // _F, ib)

        iot = lax.iota(jnp.int32, _F)

        def cb(j):
            for g in range(grp):
                lv = lane2[j, pl.ds(g * _F, _F)]
                for k in range(_F):
                    rowsv[g * _F + k] = jnp.where(iot == lv[k], 1.0, 0.0)
            pltpu.sync_copy(rowsv, ash.at[row2.at[j]], add=True)
        _fori(_NCHUNK2, cb)
        plsc.subcore_barrier()

        _fori(nzd, lambda k: pltpu.sync_copy(
            ash.at[pl.ds(tb + k * _CH, _CH)],
            out_h.at[c, pl.ds(tb + k * _CH, _CH)]))
        if rem:
            tl = pl.ds(tb + nzd * _CH, rem)
            pltpu.sync_copy(ash.at[tl], out_h.at[c, tl])

    return pl.kernel(
        body, mesh=_mesh,
        out_type=jax.ShapeDtypeStruct((_NC, n_trash, _F), jnp.float32),
        scratch_types=[
            pltpu.VMEM((_EW2,), jnp.int32),
            pltpu.VMEM((_EW2,), jnp.int32),
            pltpu.VMEM((_NCHUNK2, _CH), jnp.int32),
            pltpu.VMEM((_NCHUNK2, _CH), jnp.int32),
            pltpu.VMEM((_CH, _F), jnp.float32),
            pltpu.VMEM_SHARED((n_trash, _F), jnp.float32),
            pltpu.SemaphoreType.DMA,
        ], compiler_params=_SC_PARAMS)


def _dot(a, b):
    return jax.lax.dot_general(a, b, (((1,), (0,)), ((), ())),
                               precision=lax.Precision.HIGHEST,
                               preferred_element_type=jnp.float32)


def _blk(n):
    br = n if n <= 2500 else 2000
    return br, -(-n // br)


def _tc_update(aggp, cntp, x, w, *, emit_cnt=False, stack2=None, mlp=None):
    n = x.shape[0]
    br, nb = _blk(n)
    ix = lambda i: (i, 0)
    row16 = pl.BlockSpec((br, _F), ix)
    row32 = pl.BlockSpec((br, 2 * _F), ix)
    wsp = pl.BlockSpec((_F, _F), lambda i: (0, 0))
    bsp = pl.BlockSpec((1, _F), lambda i: (0, 0))
    wlt, bl, wrt = w

    args = [aggp[0, :n], aggp[1, :n]]
    specs = [row16, row16]
    if emit_cnt:
        args += [cntp[0][:n], cntp[1][:n]]
        specs += [row16, row16]
    else:
        args.append(cntp)
        specs.append(row16)
    args += [x, wlt, bl, wrt]
    specs += [row16, wsp, bsp, wsp]
    out_shape = [jax.ShapeDtypeStruct((n, _F), jnp.float32)]
    out_specs = [row16]
    if emit_cnt:
        out_shape.append(jax.ShapeDtypeStruct((n, _F), jnp.float32))
        out_specs.append(row16)
    if stack2 is not None:
        args.append(stack2)
        specs.append(row32)
        out_shape.append(jax.ShapeDtypeStruct((n, 2 * _F), jnp.float32))
        out_specs.append(row32)
    if mlp is not None:
        for wm, bm in mlp[:3]:
            args += [wm, bm]
            specs += [wsp, bsp]
        wf, bf = mlp[3]
        args += [wf, bf]
        specs += [pl.BlockSpec((_F, 1), lambda i: (0, 0)),
                  pl.BlockSpec((1, 1), lambda i: (0, 0))]
        out_shape.append(jax.ShapeDtypeStruct((n, 1), jnp.float32))
        out_specs.append(pl.BlockSpec((br, 1), ix))

    def kbody(*refs):
        k = 0
        a0r, a1r = refs[0], refs[1]
        k = 2
        if emit_cnt:
            cnt = refs[2][...] + refs[3][...]
            k = 4
        else:
            cnt = refs[2][...]
            k = 3
        xr, wltr, blr, wrtr = refs[k:k + 4]
        k += 4
        agg = a0r[...] + a1r[...]
        mean = agg / jnp.maximum(cnt[:, 0:1], 1.0)
        y = jnp.tanh(_dot(mean, wltr[...]) + blr[...] +
                     _dot(xr[...], wrtr[...]))
        outs = [y]
        if emit_cnt:
            outs.append(cnt)
        if stack2 is not None:
            st = refs[k][...]
            k += 1
            outs.append((jnp.concatenate([y, y], axis=1) + st) * 0.5)
        if mlp is not None:
            h = y
            for _ in range(3):
                h = jnp.tanh(_dot(h, refs[k][...]) + refs[k + 1][...])
                k += 2
            outs.append(_dot(h, refs[k][...]) + refs[k + 1][...])
            k += 2
        for o, r in zip(outs, refs[k:]):
            r[...] = o

    res = pl.pallas_call(
        kbody, grid=(nb,), in_specs=specs, out_specs=out_specs,
        out_shape=out_shape,
    )(*args)
    return res


def _tc_pool(y, n):
    y2 = y.reshape(n // 2, 2 * _F)
    br, nb = _blk(n // 2)

    def body(yr, outr):
        v = yr[...]
        outr[...] = (v[:, :_F] + v[:, _F:]) * 0.5

    return pl.pallas_call(
        body, grid=(nb,),
        in_specs=[pl.BlockSpec((br, 2 * _F), lambda i: (i, 0))],
        out_specs=pl.BlockSpec((br, _F), lambda i: (i, 0)),
        out_shape=jax.ShapeDtypeStruct((n // 2, _F), jnp.float32),
    )(y2)


def _iota_pool_mats(nc, n):
    rc = lax.broadcasted_iota(jnp.int32, (nc, n), 0)
    cc = lax.broadcasted_iota(jnp.int32, (nc, n), 1)
    P = (cc // 2 == rc).astype(jnp.float32)
    rr = lax.broadcasted_iota(jnp.int32, (n, nc), 0)
    cr = lax.broadcasted_iota(jnp.int32, (n, nc), 1)
    PT = (rr // 2 == cr).astype(jnp.float32)
    return P, PT


def _tc_deep(x3, hmat, stack2, pre_w, post_w, coarse_w):
    def body(x3r, hr, st2r, *wr):
        ws = [w[...] for w in wr[:15]]
        outr = wr[15]
        pre = [tuple(ws[0:3]), tuple(ws[3:6])]
        post = [tuple(ws[6:9]), tuple(ws[9:12])]
        co = tuple(ws[12:15])

        A = hr[...][:, :_N3]
        x = x3r[...]

        def conv(x, A, w):
            wlt, bl, wrt = w
            cnt = jnp.maximum(jnp.sum(A, axis=1, keepdims=True), 1.0)
            mean = _dot(A, x) / cnt
            return jnp.tanh(_dot(mean, wlt) + bl + _dot(x, wrt))

        stacks = []
        amats = []
        for l in range(3, _NLVL):
            n = _LVL[l]
            for i in range(2):
                x = conv(x, A, pre[i])
            stacks.append(x)
            amats.append(A)
            nc = (n + 1) // 2
            P, PT = _iota_pool_mats(nc, n)
            sizes = jnp.where(
                lax.broadcasted_iota(jnp.int32, (nc, 1), 0) * 2 + 1 < n,
                2.0, 1.0)
            x = _dot(P, x) / sizes
            An = _dot(_dot(P, A), PT)
            ri = lax.broadcasted_iota(jnp.int32, (nc, nc), 0)
            ci = lax.broadcasted_iota(jnp.int32, (nc, nc), 1)
            A = jnp.where(ri == ci, 0.0, An)
        x = conv(x, A, co)
        for l in range(_NLVL - 1, 2, -1):
            n = _LVL[l]
            nc = (n + 1) // 2
            _, PT = _iota_pool_mats(nc, n)
            x = (_dot(PT, x) + stacks.pop()) * 0.5
            A = amats.pop()
            for i in range(2):
                x = conv(x, A, post[i])
        outr[...] = (jnp.concatenate([x, x], axis=1) + st2r[...]) * 0.5

    hw = _SEG3 * _F

    def full(s):
        return pl.BlockSpec(s, lambda: tuple(0 for _ in s))

    wargs = []
    wspecs = []
    for trip in (*pre_w, *post_w, coarse_w):
        for a in trip:
            wargs.append(a)
            wspecs.append(full(a.shape))
    return pl.pallas_call(
        body,
        in_specs=[full((_N3, _F)), full((_N3, hw)),
                  full((_N3, 2 * _F))] + wspecs,
        out_specs=full((_N3, 2 * _F)),
        out_shape=jax.ShapeDtypeStruct((_N3, 2 * _F), jnp.float32),
    )(x3, hmat, stack2, *wargs)


def _prep_w(tr):
    wl, bl, wr = tr
    return (wl.T, bl.reshape(1, _F), wr.T)


def kernel(x, params, edge_index, batch):
    src = edge_index[0]
    dst = edge_index[1]
    x16 = jnp.pad(x, ((0, 0), (0, _F - 1)))

    wl_f, bl_f, wr_f = params['first']
    first_w = (jnp.pad(wl_f.T, ((0, _F - 1), (0, 0))),
               bl_f.reshape(1, _F),
               jnp.pad(wr_f.T, ((0, _F - 1), (0, 0))))
    pre_w = [_prep_w(t) for t in params['pre']]
    post_w = [_prep_w(t) for t in params['post']]
    coarse_w = _prep_w(params['coarse'])
    mlp_w = [(params[nm][0].T, params[nm][1].reshape(1, _F))
             for nm in ('lin1', 'lin2', 'lin3')]
    wf, bf = params['final']
    mlp_w.append((wf.T, bf.reshape(1, 1)))

    aggk = {l: _mk_sc_agg(l, _LVL[l], False) for l in (0, 1, 2)}
    aggck = {l: _mk_sc_agg(l, _LVL[l], True) for l in (0, 1, 2)}

    hist = _mk_sc_hist()(src, dst)
    hmat = jnp.concatenate(
        [hist[0, :_HHALF], hist[1, :_HHALF]], axis=0
    ).reshape(_N3, _SEG3 * _F)

    aggp, cntp = aggck[0](src, dst, x16)
    cur, cnt0 = _tc_update(aggp, cntp, x16, first_w, emit_cnt=True)

    stacks = []
    cnts = {0: cnt0}
    for l in range(3):
        n = _LVL[l]
        for i in range(2):
            if l > 0 and i == 0:
                aggp, cntp = aggck[l](src, dst, cur)
                cur, cnts[l] = _tc_update(aggp, cntp, cur, pre_w[i],
                                          emit_cnt=True)
            else:
                aggp = aggk[l](src, dst, cur)
                (cur,) = _tc_update(aggp, cnts[l], cur, pre_w[i])
        stacks.append(cur)
        cur = _tc_pool(cur, n)

    up = _tc_deep(cur, hmat, stacks[2].reshape(_N3, 2 * _F),
                  pre_w, post_w, coarse_w)
    cur = up.reshape(_LVL[2], _F)

    for l in (2, 1, 0):
        aggp = aggk[l](src, dst, cur)
        (cur,) = _tc_update(aggp, cnts[l], cur, post_w[0])
        aggp = aggk[l](src, dst, cur)
        if l > 0:
            st2 = stacks[l - 1].reshape(_LVL[l], 2 * _F)
            _, up = _tc_update(aggp, cnts[l], cur, post_w[1], stack2=st2)
            cur = up.reshape(_LVL[l - 1], _F)
        else:
            _, out = _tc_update(aggp, cnts[l], cur, post_w[1], mlp=mlp_w)
    return out
